# baseline (speedup 1.0000x reference)
import functools

import jax
import jax.numpy as jnp
from jax import lax
from jax.experimental import pallas as pl
from jax.experimental.pallas import tpu as pltpu
from jax.experimental.pallas import tpu_sc as plsc

_NC = 2
_NS = 16
_NW = _NC * _NS
_CHUNK = 128


def _build_gather(B, D):
    b_per_w = B // _NW
    nch = b_per_w // _CHUNK
    mesh = plsc.VectorSubcoreMesh(core_axis_name="c", subcore_axis_name="s")

    @functools.partial(
        pl.kernel,
        mesh=mesh,
        out_type=(
            jax.ShapeDtypeStruct((B, D), jnp.float32),
            jax.ShapeDtypeStruct((B, D), jnp.float32),
        ),
        scratch_types=[
            pltpu.VMEM((nch, _CHUNK), jnp.int32),
            pltpu.VMEM((nch, _CHUNK), jnp.int32),
            pltpu.VMEM((b_per_w, D), jnp.float32),
            pltpu.VMEM((b_per_w, D), jnp.float32),
            pltpu.SemaphoreType.DMA,
            pltpu.SemaphoreType.DMA,
        ],
        compiler_params=pltpu.CompilerParams(use_tc_tiling_on_sc=False),
    )
    def gather(xu, xb, uemb, bemb, out_u, out_b,
               idx_u, idx_b, rows_u, rows_b, sem_u, sem_b):
        wid = lax.axis_index("s") * _NC + lax.axis_index("c")
        base = wid * b_per_w
        pltpu.sync_copy(xu.at[wid], idx_u)
        pltpu.sync_copy(xb.at[wid], idx_b)
        copies = []
        for c in range(nch):
            sl = pl.ds(c * _CHUNK, _CHUNK)
            copies.append(pltpu.async_copy(uemb.at[idx_u.at[c]], rows_u.at[sl], sem_u))
            copies.append(pltpu.async_copy(bemb.at[idx_b.at[c]], rows_b.at[sl], sem_b))
        for cp in copies:
            cp.wait()
        pltpu.sync_copy(rows_u, out_u.at[pl.ds(base, b_per_w)])
        pltpu.sync_copy(rows_b, out_b.at[pl.ds(base, b_per_w)])

    return gather


def _mlp_body(u, b, w1u, w1b, fcb, w2, hlb, out):
    h = (jnp.dot(u[...], w1u[...], preferred_element_type=jnp.float32)
         + jnp.dot(b[...], w1b[...], preferred_element_type=jnp.float32)
         + fcb[...])
    h2 = jnp.dot(h, w2[...], preferred_element_type=jnp.float32) + hlb[...]
    out[...] = 1.0 / (1.0 + jnp.exp(-h2)) * 4.0 + 1.0


def _mlp(u, b, fc_w, fc_b, hl_w, hl_b, blk=2048):
    B, D = u.shape
    H = fc_w.shape[1]
    O = hl_w.shape[1]
    grid = B // blk
    return pl.pallas_call(
        _mlp_body,
        grid=(grid,),
        in_specs=[
            pl.BlockSpec((blk, D), lambda i: (i, 0)),
            pl.BlockSpec((blk, D), lambda i: (i, 0)),
            pl.BlockSpec((D, H), lambda i: (0, 0)),
            pl.BlockSpec((D, H), lambda i: (0, 0)),
            pl.BlockSpec((1, H), lambda i: (0, 0)),
            pl.BlockSpec((H, O), lambda i: (0, 0)),
            pl.BlockSpec((1, O), lambda i: (0, 0)),
        ],
        out_specs=pl.BlockSpec((blk, O), lambda i: (i, 0)),
        out_shape=jax.ShapeDtypeStruct((B, O), jnp.float32),
    )(u, b, fc_w[:D], fc_w[D:], fc_b.reshape(1, H), hl_w, hl_b.reshape(1, O))


def kernel(x, user_emb, book_emb, fc_w, fc_b, hl_w, hl_b):
    B = x.shape[0]
    D = user_emb.shape[1]
    b_per_w = B // _NW
    nch = b_per_w // _CHUNK
    xu = x[:, 0].reshape(_NW, nch, _CHUNK)
    xb = x[:, 1].reshape(_NW, nch, _CHUNK)
    u_rows, b_rows = _build_gather(B, D)(xu, xb, user_emb, book_emb)
    return _mlp(u_rows, b_rows, fc_w, fc_b, hl_w, hl_b)


# speedup vs baseline: 1.5466x; 1.5466x over previous
import functools

import jax
import jax.numpy as jnp
from jax import lax
from jax.experimental import pallas as pl
from jax.experimental.pallas import tpu as pltpu
from jax.experimental.pallas import tpu_sc as plsc

_NC = 2
_NS = 16
_NW = _NC * _NS
_GRP = 64


def _build_gather(B, D):
    b_per_w = B // _NW
    ngrp = b_per_w // _GRP
    mesh = plsc.VectorSubcoreMesh(core_axis_name="c", subcore_axis_name="s")

    @functools.partial(
        pl.kernel,
        mesh=mesh,
        out_type=(
            jax.ShapeDtypeStruct((B, D), jnp.float32),
            jax.ShapeDtypeStruct((B, D), jnp.float32),
        ),
        scratch_types=[
            pltpu.VMEM((ngrp, _GRP), jnp.int32),
            pltpu.VMEM((ngrp, _GRP), jnp.int32),
            pltpu.VMEM((_GRP, D), jnp.float32),
            pltpu.VMEM((_GRP, D), jnp.float32),
            pltpu.SemaphoreType.DMA,
            pltpu.SemaphoreType.DMA,
        ],
        compiler_params=pltpu.CompilerParams(use_tc_tiling_on_sc=True),
    )
    def gather(xu3, xb3, tu, tb, out_u, out_b,
               idx_u, idx_b, rows_u, rows_b, sem_u, sem_b):
        wid = lax.axis_index("s") * _NC + lax.axis_index("c")
        base = wid * b_per_w
        pltpu.sync_copy(xu3.at[wid], idx_u)
        pltpu.sync_copy(xb3.at[wid], idx_b)

        def group_body(g, _):
            copies = []
            for k in range(_GRP // 16):
                vu = idx_u[g, pl.ds(k * 16, 16)]
                vb = idx_b[g, pl.ds(k * 16, 16)]
                for j in range(16):
                    t = k * 16 + j
                    copies.append(pltpu.async_copy(
                        tu.at[pl.ds(vu[j], 1), :], rows_u.at[pl.ds(t, 1), :], sem_u))
                    copies.append(pltpu.async_copy(
                        tb.at[pl.ds(vb[j], 1), :], rows_b.at[pl.ds(t, 1), :], sem_b))
            for cp in copies:
                cp.wait()
            row0 = pl.multiple_of(base + g * _GRP, _GRP)
            pltpu.sync_copy(rows_u, out_u.at[pl.ds(row0, _GRP)])
            pltpu.sync_copy(rows_b, out_b.at[pl.ds(row0, _GRP)])
            return _

        lax.fori_loop(0, ngrp, group_body, None)

    return gather


def _mlp_body(u, b, w1u, w1b, fcb, w2, hlb, out):
    h = (jnp.dot(u[...], w1u[...], preferred_element_type=jnp.float32)
         + jnp.dot(b[...], w1b[...], preferred_element_type=jnp.float32)
         + fcb[...])
    h2 = jnp.dot(h, w2[...], preferred_element_type=jnp.float32) + hlb[...]
    out[...] = 1.0 / (1.0 + jnp.exp(-h2)) * 4.0 + 1.0


def _mlp(u, b, fc_w, fc_b, hl_w, hl_b, blk=2048):
    B, D = u.shape
    H = fc_w.shape[1]
    O = hl_w.shape[1]
    grid = B // blk
    return pl.pallas_call(
        _mlp_body,
        grid=(grid,),
        in_specs=[
            pl.BlockSpec((blk, D), lambda i: (i, 0)),
            pl.BlockSpec((blk, D), lambda i: (i, 0)),
            pl.BlockSpec((D, H), lambda i: (0, 0)),
            pl.BlockSpec((D, H), lambda i: (0, 0)),
            pl.BlockSpec((1, H), lambda i: (0, 0)),
            pl.BlockSpec((H, O), lambda i: (0, 0)),
            pl.BlockSpec((1, O), lambda i: (0, 0)),
        ],
        out_specs=pl.BlockSpec((blk, O), lambda i: (i, 0)),
        out_shape=jax.ShapeDtypeStruct((B, O), jnp.float32),
    )(u, b, fc_w[:D], fc_w[D:], fc_b.reshape(1, H), hl_w, hl_b.reshape(1, O))


def kernel(x, user_emb, book_emb, fc_w, fc_b, hl_w, hl_b):
    B = x.shape[0]
    D = user_emb.shape[1]
    b_per_w = B // _NW
    ngrp = b_per_w // _GRP
    xu3 = x[:, 0].reshape(_NW, ngrp, _GRP)
    xb3 = x[:, 1].reshape(_NW, ngrp, _GRP)
    u_rows, b_rows = _build_gather(B, D)(xu3, xb3, user_emb, book_emb)
    return _mlp(u_rows, b_rows, fc_w, fc_b, hl_w, hl_b)
